# baseline (speedup 1.0000x reference)
import numpy as np
import jax
import jax.numpy as jnp
from jax.experimental import pallas as pl

_EXTRA_IDXS = np.array([
    9120, 9929, 9448, 616, 6,
    5770, 5780, 8846, 8463, 8474, 8635,
    5361, 4933, 5058, 5169, 5286,
    8079, 7669, 7794, 7905, 8022
], dtype=np.int32)

_B, _V, _C = 1024, 10475, 3
_J, _E = 55, 21


def _body(*refs):
    jt_ref = refs[0]
    vrefs = refs[1:1 + _E]
    out_ref = refs[1 + _E]
    out_ref[:, 0:_J, :] = jt_ref[:]
    for j in range(_E):
        s = int(_EXTRA_IDXS[j]) % 8
        out_ref[:, _J + j:_J + j + 1, :] = vrefs[j][:, s:s + 1, :]


@jax.jit
def kernel(vertices, joints):
    vt = jnp.transpose(vertices, (2, 1, 0))
    jt = jnp.transpose(joints, (2, 1, 0))
    lb = _B // 2
    in_specs = [pl.BlockSpec((1, _J, lb), lambda c, i: (c, 0, i))]
    for j in range(_E):
        blk = int(_EXTRA_IDXS[j]) // 8
        in_specs.append(
            pl.BlockSpec((1, 8, lb), lambda c, i, _blk=blk: (c, _blk, i)))
    out_t = pl.pallas_call(
        _body,
        grid=(_C, 2),
        in_specs=in_specs,
        out_specs=pl.BlockSpec((1, _J + _E, lb), lambda c, i: (c, 0, i)),
        out_shape=jax.ShapeDtypeStruct((_C, _J + _E, _B), jnp.float32),
    )(jt, *([vt] * _E))
    return jnp.transpose(out_t, (2, 1, 0))


# speedup vs baseline: 1.8253x; 1.8253x over previous
import numpy as np
import jax
import jax.numpy as jnp
from jax.experimental import pallas as pl

_EXTRA_IDXS = np.array([
    9120, 9929, 9448, 616, 6,
    5770, 5780, 8846, 8463, 8474, 8635,
    5361, 4933, 5058, 5169, 5286,
    8079, 7669, 7794, 7905, 8022
], dtype=np.int32)

_B, _V, _C = 1024, 10475, 3
_J, _E = 55, 21


def _body(*refs):
    jt_ref = refs[0]
    vrefs = refs[1:1 + _E]
    out_ref = refs[1 + _E]
    out_ref[:, 0:_J, :] = jt_ref[:]
    for j in range(_E):
        s = int(_EXTRA_IDXS[j]) % 8
        out_ref[:, _J + j:_J + j + 1, :] = vrefs[j][:, s:s + 1, :]


@jax.jit
def kernel(vertices, joints):
    vt = jnp.transpose(vertices, (2, 1, 0))
    jt = jnp.transpose(joints, (2, 1, 0))
    lb = _B // 2
    in_specs = [pl.BlockSpec((_C, _J, lb), lambda i: (0, 0, i))]
    for j in range(_E):
        blk = int(_EXTRA_IDXS[j]) // 8
        in_specs.append(
            pl.BlockSpec((_C, 8, lb), lambda i, _blk=blk: (0, _blk, i)))
    out_t = pl.pallas_call(
        _body,
        grid=(2,),
        in_specs=in_specs,
        out_specs=pl.BlockSpec((_C, _J + _E, lb), lambda i: (0, 0, i)),
        out_shape=jax.ShapeDtypeStruct((_C, _J + _E, _B), jnp.float32),
    )(jt, *([vt] * _E))
    return jnp.transpose(out_t, (2, 1, 0))
